# baseline (speedup 1.0000x reference)
import functools

import jax
import jax.numpy as jnp
from jax import lax
from jax.experimental import pallas as pl
from jax.experimental.pallas import tpu as pltpu
from jax.experimental.pallas import tpu_sc as plsc

EMB = 32
L = 16
NC = 2
NS = 16
NW = NC * NS
CHUNK = 128


def _sc_body(batch, row_id2, col_id2, row_emb, col_emb, row_bias, col_bias,
             gb16, out_hbm, ridx_v, cidx_v, rrows_v, crows_v, rb_v, cb_v,
             gb_v, out_v, sem):
    bpw = batch // NW
    nchunk = bpw // CHUNK
    wid = lax.axis_index("s") * NC + lax.axis_index("c")
    base = wid * bpw

    pltpu.sync_copy(row_id2.at[pl.ds(wid * nchunk, nchunk)], ridx_v)
    pltpu.sync_copy(col_id2.at[pl.ds(wid * nchunk, nchunk)], cidx_v)
    pltpu.sync_copy(gb16, gb_v)

    copies = []
    for j in range(nchunk):
        s = pl.ds(j * CHUNK, CHUNK)
        copies.append(pltpu.async_copy(row_emb.at[ridx_v.at[j]], rrows_v.at[s], sem))
        copies.append(pltpu.async_copy(col_emb.at[cidx_v.at[j]], crows_v.at[s], sem))
        copies.append(pltpu.async_copy(row_bias.at[ridx_v.at[j]], rb_v.at[s], sem))
        copies.append(pltpu.async_copy(col_bias.at[cidx_v.at[j]], cb_v.at[s], sem))
    for c in copies:
        c.wait()

    gvec = gb_v[...]
    lane = lax.iota(jnp.int32, L)

    def block(i, _):
        b0 = i * L
        idx_b = lane + b0
        acc = rb_v[pl.ds(b0, L)] + cb_v[pl.ds(b0, L)] + gvec
        for e in range(EMB):
            e_idx = jnp.full((L,), e, jnp.int32)
            rv = plsc.load_gather(rrows_v, [idx_b, e_idx])
            cv = plsc.load_gather(crows_v, [idx_b, e_idx])
            acc += rv * cv
        out_v[pl.ds(b0, L)] = acc
        return ()

    lax.fori_loop(0, bpw // L, block, ())

    pltpu.sync_copy(out_v, out_hbm.at[pl.ds(base, bpw)])


@functools.partial(jax.jit, static_argnames=("batch",))
def _mf_sc(row_id2, col_id2, row_emb, col_emb, row_bias, col_bias, gb16,
           *, batch):
    bpw = batch // NW
    mesh = plsc.VectorSubcoreMesh(core_axis_name="c", subcore_axis_name="s")
    return pl.kernel(
        functools.partial(_sc_body, batch),
        out_type=jax.ShapeDtypeStruct((batch,), jnp.float32),
        mesh=mesh,
        scratch_types=[
            pltpu.VMEM((bpw // CHUNK, CHUNK), jnp.int32),
            pltpu.VMEM((bpw // CHUNK, CHUNK), jnp.int32),
            pltpu.VMEM((bpw, EMB), jnp.float32),
            pltpu.VMEM((bpw, EMB), jnp.float32),
            pltpu.VMEM((bpw,), jnp.float32),
            pltpu.VMEM((bpw,), jnp.float32),
            pltpu.VMEM((L,), jnp.float32),
            pltpu.VMEM((bpw,), jnp.float32),
            pltpu.SemaphoreType.DMA,
        ],
        compiler_params=pltpu.CompilerParams(
            needs_layout_passes=False, use_tc_tiling_on_sc=False),
    )(row_id2, col_id2, row_emb, col_emb, row_bias, col_bias, gb16)


def kernel(row_id, col_id, row_emb_table, col_emb_table, row_bias_table,
           col_bias_table, global_bias):
    batch = row_id.shape[0]
    row_id2 = row_id.astype(jnp.int32).reshape(batch // CHUNK, CHUNK)
    col_id2 = col_id.astype(jnp.int32).reshape(batch // CHUNK, CHUNK)
    rb_flat = row_bias_table.reshape(-1)
    cb_flat = col_bias_table.reshape(-1)
    gb16 = jnp.broadcast_to(jnp.reshape(global_bias, (1,)), (L,))
    out = _mf_sc(row_id2, col_id2, row_emb_table, col_emb_table, rb_flat,
                 cb_flat, gb16, batch=batch)
    return out.reshape(batch, 1)
